# Initial kernel scaffold; baseline (speedup 1.0000x reference)
import jax
import jax.numpy as jnp
from jax.experimental import pallas as pl


def kernel(hidden_states, gate_weight):
    raise NotImplementedError("write your pallas kernel here")


# speedup vs baseline: 1.0100x; 1.0100x over previous
import functools

import jax
import jax.numpy as jnp
from jax.experimental import pallas as pl
from jax.experimental.pallas import tpu as pltpu

B, L, D = 2, 2048, 4096
N = 64
K = 8
ALPHA = 0.001
BT = 512
T = B * L


def _gate_kernel(x_ref, wt_ref, probs_ref, sel_ref, w_ref, stats_ref):
    step = pl.program_id(0)

    @pl.when(step == 0)
    def _init():
        stats_ref[...] = jnp.zeros_like(stats_ref)

    logits = jnp.dot(x_ref[...], wt_ref[...], preferred_element_type=jnp.float32)
    m = jnp.max(logits, axis=-1, keepdims=True)
    e = jnp.exp(logits - m)
    z = jnp.sum(e, axis=-1, keepdims=True)
    probs = e / z
    probs_ref[...] = probs

    iota = jax.lax.broadcasted_iota(jnp.int32, probs.shape, 1)
    cur = probs
    sel_onehot = jnp.zeros_like(probs)
    idx_cols = []
    val_cols = []
    for _ in range(K):
        mx = jnp.max(cur, axis=-1, keepdims=True)
        eq = cur == mx
        idx = jnp.min(jnp.where(eq, iota, N), axis=-1, keepdims=True)
        oh = (iota == idx).astype(probs.dtype)
        sel_onehot = sel_onehot + oh
        idx_cols.append(idx)
        val_cols.append(mx)
        cur = jnp.where(oh > 0, -1.0, cur)
    sel_ref[...] = jnp.concatenate(idx_cols, axis=1)
    vals = jnp.concatenate(val_cols, axis=1)
    w_ref[...] = vals / jnp.sum(vals, axis=-1, keepdims=True)

    stats_ref[2:3, :] += jnp.sum(sel_onehot, axis=0, keepdims=True)
    stats_ref[3:4, :] += jnp.sum(probs, axis=0, keepdims=True)

    @pl.when(step == pl.num_programs(0) - 1)
    def _finalize():
        counts = stats_ref[2:3, :]
        psum = stats_ref[3:4, :]
        f_i = counts / (T * K)
        p_i = psum / T
        stats_ref[1:2, 0:1] = ALPHA * N * jnp.sum(f_i * p_i, keepdims=True)
        stats_ref[0:1, :] = counts / jnp.sum(counts)


@functools.partial(jax.jit, static_argnames=())
def kernel(hidden_states, gate_weight):
    x = hidden_states.reshape(T, D)
    wt = gate_weight.T
    grid = (T // BT,)
    probs, sel, w, stats = pl.pallas_call(
        _gate_kernel,
        grid=grid,
        in_specs=[
            pl.BlockSpec((BT, D), lambda i: (i, 0)),
            pl.BlockSpec((D, N), lambda i: (0, 0)),
        ],
        out_specs=[
            pl.BlockSpec((BT, N), lambda i: (i, 0)),
            pl.BlockSpec((BT, K), lambda i: (i, 0)),
            pl.BlockSpec((BT, K), lambda i: (i, 0)),
            pl.BlockSpec((8, N), lambda i: (0, 0)),
        ],
        out_shape=[
            jax.ShapeDtypeStruct((T, N), jnp.float32),
            jax.ShapeDtypeStruct((T, K), jnp.int32),
            jax.ShapeDtypeStruct((T, K), jnp.float32),
            jax.ShapeDtypeStruct((8, N), jnp.float32),
        ],
    )(x, wt)
    router_probs = probs.reshape(B, L, N)
    selected_experts = sel.reshape(B, L, K)
    expert_weights = w.reshape(B, L, K)
    aux_loss = stats[1, 0]
    expert_usage = stats[0]
    return (router_probs, selected_experts, expert_weights, aux_loss, expert_usage)
